# baseline (speedup 1.0000x reference)
import functools

import jax
import jax.numpy as jnp
from jax import lax
from jax.experimental import pallas as pl
from jax.experimental.pallas import tpu as pltpu
from jax.experimental.pallas import tpu_sc as plsc

_L = 16
_NS = 16
_EPS = 1e-5
_LN2 = 0.6931471805599453


def _vlog(x):
    bits = lax.bitcast_convert_type(x, jnp.int32)
    e = (bits >> 23) - 127
    m = lax.bitcast_convert_type((bits & 0x007FFFFF) | 0x3F800000, jnp.float32)
    t = (m - 1.0) / (m + 1.0)
    t2 = t * t
    p = 2.0 * t * (1.0 + t2 * (1.0 / 3.0 + t2 * (0.2 + t2 * (1.0 / 7.0
                   + t2 * (1.0 / 9.0)))))
    return e.astype(jnp.float32) * _LN2 + p


def _term(tau, dtv):
    return -_vlog(tau) - (dtv + _EPS) / tau


def _make_sc_call(n, nseg, denom):
    chunk = n // _NS
    mesh = plsc.VectorSubcoreMesh(
        core_axis_name="c", subcore_axis_name="s", num_cores=1,
        num_subcores=_NS)

    @functools.partial(
        pl.kernel,
        out_type=jax.ShapeDtypeStruct((1,), jnp.float32),
        mesh=mesh,
        compiler_params=pltpu.CompilerParams(needs_layout_passes=False),
        scratch_types=dict(
            tau_v=pltpu.VMEM((chunk,), jnp.float32),
            dt_v=pltpu.VMEM((chunk,), jnp.float32),
            off_v=pltpu.VMEM((_L,), jnp.int32),
            ilo_v=pltpu.VMEM((_L,), jnp.int32),
            ihi_v=pltpu.VMEM((_L,), jnp.int32),
            g_tlo=pltpu.VMEM((_L,), jnp.float32),
            g_thi=pltpu.VMEM((_L,), jnp.float32),
            g_dlo=pltpu.VMEM((_L,), jnp.float32),
            g_dhi=pltpu.VMEM((_L,), jnp.float32),
            acc_v=pltpu.VMEM((_L,), jnp.float32),
            red_v=pltpu.VMEM((_NS * _L,), jnp.float32),
            out_v=pltpu.VMEM((1,), jnp.float32),
            shared=pltpu.VMEM_SHARED((_NS * _L,), jnp.float32),
            sem_chunk=pltpu.SemaphoreType.DMA,
            sem_se=pltpu.SemaphoreType.DMA,
            sem_g=pltpu.SemaphoreType.DMA,
        ),
    )
    def sc_loss(tau_hbm, dt_hbm, off_hbm, out_hbm, *, tau_v, dt_v, off_v,
                ilo_v, ihi_v, g_tlo, g_thi, g_dlo, g_dhi, acc_v, red_v,
                out_v, shared, sem_chunk, sem_se, sem_g):
        sid = lax.axis_index("s")
        is_w0 = sid == 0

        base = sid * chunk
        c_tau = pltpu.async_copy(tau_hbm.at[pl.ds(base, chunk)], tau_v,
                                 sem_chunk)
        c_dt = pltpu.async_copy(dt_hbm.at[pl.ds(base, chunk)], dt_v,
                                sem_chunk)

        @pl.when(is_w0)
        def _prefetch_offsets():
            pltpu.async_copy(off_hbm, off_v.at[pl.ds(0, nseg + 1)], sem_se)

        c_tau.wait()
        c_dt.wait()

        @plsc.parallel_loop(0, chunk, step=_L, unroll=2,
                            carry=jnp.zeros((_L,), jnp.float32))
        def _dense(k, acc):
            tv = tau_v[pl.ds(k, _L)]
            dv = dt_v[pl.ds(k, _L)]
            return acc + _term(tv, dv)

        acc_v[...] = _dense

        def _bounds():
            iot = lax.iota(jnp.int32, _L)
            sv = plsc.load_gather(off_v, [jnp.minimum(iot, nseg)])
            ev = plsc.load_gather(off_v, [jnp.minimum(iot + 1, nseg)])
            return sv, ev

        @pl.when(is_w0)
        def _fire_gathers():
            pltpu.make_async_copy(off_hbm, off_v.at[pl.ds(0, nseg + 1)],
                                  sem_se).wait()
            sv, ev = _bounds()
            ilo_v[...] = jnp.minimum(sv, n - 1)
            ihi_v[...] = jnp.maximum(ev - 1, 0)
            pltpu.async_copy(tau_hbm.at[ilo_v], g_tlo, sem_g)
            pltpu.async_copy(tau_hbm.at[ihi_v], g_thi, sem_g)
            pltpu.async_copy(dt_hbm.at[ilo_v], g_dlo, sem_g)
            pltpu.async_copy(dt_hbm.at[ihi_v], g_dhi, sem_g)

        @pl.when(jnp.logical_not(is_w0))
        def _stage():
            pltpu.sync_copy(acc_v, shared.at[pl.ds(sid * _L, _L)])

        plsc.subcore_barrier()

        @pl.when(is_w0)
        def _finalize():
            pltpu.sync_copy(shared.at[pl.ds(_L, (_NS - 1) * _L)],
                            red_v.at[pl.ds(_L, (_NS - 1) * _L)])
            tot = acc_v[...]
            for r in range(1, _NS):
                tot = tot + red_v[pl.ds(r * _L, _L)]
            pltpu.make_async_copy(tau_hbm.at[ilo_v], g_tlo, sem_g).wait()
            pltpu.make_async_copy(tau_hbm.at[ihi_v], g_thi, sem_g).wait()
            pltpu.make_async_copy(dt_hbm.at[ilo_v], g_dlo, sem_g).wait()
            pltpu.make_async_copy(dt_hbm.at[ihi_v], g_dhi, sem_g).wait()
            sv, ev = _bounds()
            zero = jnp.zeros((_L,), jnp.float32)
            mlo = ev > sv
            mhi = (ev - sv) >= 2
            corr = (jnp.where(mlo, _term(g_tlo[...], g_dlo[...]), zero)
                    + jnp.where(mhi, _term(g_thi[...], g_dhi[...]), zero))
            final = jnp.sum(corr - tot) * (1.0 / denom)
            lane0 = lax.iota(jnp.int32, _L) == 0
            plsc.store_scatter(out_v, [jnp.zeros((_L,), jnp.int32)],
                               jnp.broadcast_to(final, (_L,)), mask=lane0)
            pltpu.sync_copy(out_v, out_hbm)

    return sc_loss


def kernel(pred_next_dt, next_dt, dt, offsets):
    tau = lax.stop_gradient(pred_next_dt)
    n = tau.shape[0]
    nseg = offsets.shape[0] - 1
    out = _make_sc_call(n, nseg, float(offsets.shape[0]))(
        tau, dt[:, 0], offsets)
    loss = out.reshape(())
    return (loss, loss)


# speedup vs baseline: 1.0031x; 1.0031x over previous
import functools

import jax
import jax.numpy as jnp
from jax import lax
from jax.experimental import pallas as pl
from jax.experimental.pallas import tpu as pltpu
from jax.experimental.pallas import tpu_sc as plsc

_L = 16
_NS = 16
_EPS = 1e-5
_LN2 = 0.6931471805599453


def _vlog(x):
    bits = lax.bitcast_convert_type(x, jnp.int32)
    e = (bits >> 23) - 127
    m = lax.bitcast_convert_type((bits & 0x007FFFFF) | 0x3F800000, jnp.float32)
    t = (m - 1.0) / (m + 1.0)
    t2 = t * t
    p = 2.0 * t * (1.0 + t2 * (1.0 / 3.0 + t2 * (0.2 + t2 * (1.0 / 7.0
                   + t2 * (1.0 / 9.0)))))
    return e.astype(jnp.float32) * _LN2 + p


def _term(tau, dtv):
    return -_vlog(tau) - (dtv + _EPS) / tau


def _make_sc_call(n, nseg, denom):
    chunk = n // _NS
    mesh = plsc.VectorSubcoreMesh(
        core_axis_name="c", subcore_axis_name="s", num_cores=1,
        num_subcores=_NS)

    @functools.partial(
        pl.kernel,
        out_type=jax.ShapeDtypeStruct((1,), jnp.float32),
        mesh=mesh,
        compiler_params=pltpu.CompilerParams(needs_layout_passes=False),
        scratch_types=dict(
            tau_v=pltpu.VMEM((chunk,), jnp.float32),
            dt_v=pltpu.VMEM((chunk,), jnp.float32),
            off_v=pltpu.VMEM((_L,), jnp.int32),
            ilo_v=pltpu.VMEM((_L,), jnp.int32),
            ihi_v=pltpu.VMEM((_L,), jnp.int32),
            g_tlo=pltpu.VMEM((_L,), jnp.float32),
            g_thi=pltpu.VMEM((_L,), jnp.float32),
            g_dlo=pltpu.VMEM((_L,), jnp.float32),
            g_dhi=pltpu.VMEM((_L,), jnp.float32),
            acc_v=pltpu.VMEM((_L,), jnp.float32),
            red_v=pltpu.VMEM((_NS * _L,), jnp.float32),
            out_v=pltpu.VMEM((1,), jnp.float32),
            shared=pltpu.VMEM_SHARED((_NS * _L,), jnp.float32),
            sem_chunk=pltpu.SemaphoreType.DMA,
            sem_se=pltpu.SemaphoreType.DMA,
            sem_g=pltpu.SemaphoreType.DMA,
        ),
    )
    def sc_loss(tau_hbm, dt_hbm, off_hbm, out_hbm, *, tau_v, dt_v, off_v,
                ilo_v, ihi_v, g_tlo, g_thi, g_dlo, g_dhi, acc_v, red_v,
                out_v, shared, sem_chunk, sem_se, sem_g):
        sid = lax.axis_index("s")
        is_w0 = sid == 0

        base = sid * chunk
        c_tau = pltpu.async_copy(tau_hbm.at[pl.ds(base, chunk)], tau_v,
                                 sem_chunk)
        c_dt = pltpu.async_copy(dt_hbm.at[pl.ds(base, chunk)], dt_v,
                                sem_chunk)

        @pl.when(is_w0)
        def _prefetch_offsets():
            pltpu.async_copy(off_hbm, off_v.at[pl.ds(0, nseg + 1)], sem_se)

        c_tau.wait()
        c_dt.wait()

        @plsc.parallel_loop(0, chunk, step=_L, unroll=4,
                            carry=jnp.zeros((_L,), jnp.float32))
        def _dense(k, acc):
            tv = tau_v[pl.ds(k, _L)]
            dv = dt_v[pl.ds(k, _L)]
            return acc + _term(tv, dv)

        acc_v[...] = _dense

        def _bounds():
            iot = lax.iota(jnp.int32, _L)
            sv = plsc.load_gather(off_v, [jnp.minimum(iot, nseg)])
            ev = plsc.load_gather(off_v, [jnp.minimum(iot + 1, nseg)])
            return sv, ev

        @pl.when(is_w0)
        def _fire_gathers():
            pltpu.make_async_copy(off_hbm, off_v.at[pl.ds(0, nseg + 1)],
                                  sem_se).wait()
            sv, ev = _bounds()
            ilo_v[...] = jnp.minimum(sv, n - 1)
            ihi_v[...] = jnp.maximum(ev - 1, 0)
            pltpu.async_copy(tau_hbm.at[ilo_v], g_tlo, sem_g)
            pltpu.async_copy(tau_hbm.at[ihi_v], g_thi, sem_g)
            pltpu.async_copy(dt_hbm.at[ilo_v], g_dlo, sem_g)
            pltpu.async_copy(dt_hbm.at[ihi_v], g_dhi, sem_g)

        @pl.when(jnp.logical_not(is_w0))
        def _stage():
            pltpu.sync_copy(acc_v, shared.at[pl.ds(sid * _L, _L)])

        plsc.subcore_barrier()

        @pl.when(is_w0)
        def _finalize():
            pltpu.sync_copy(shared.at[pl.ds(_L, (_NS - 1) * _L)],
                            red_v.at[pl.ds(_L, (_NS - 1) * _L)])
            tot = acc_v[...]
            for r in range(1, _NS):
                tot = tot + red_v[pl.ds(r * _L, _L)]
            pltpu.make_async_copy(tau_hbm.at[ilo_v], g_tlo, sem_g).wait()
            pltpu.make_async_copy(tau_hbm.at[ihi_v], g_thi, sem_g).wait()
            pltpu.make_async_copy(dt_hbm.at[ilo_v], g_dlo, sem_g).wait()
            pltpu.make_async_copy(dt_hbm.at[ihi_v], g_dhi, sem_g).wait()
            sv, ev = _bounds()
            zero = jnp.zeros((_L,), jnp.float32)
            mlo = ev > sv
            mhi = (ev - sv) >= 2
            corr = (jnp.where(mlo, _term(g_tlo[...], g_dlo[...]), zero)
                    + jnp.where(mhi, _term(g_thi[...], g_dhi[...]), zero))
            final = jnp.sum(corr - tot) * (1.0 / denom)
            lane0 = lax.iota(jnp.int32, _L) == 0
            plsc.store_scatter(out_v, [jnp.zeros((_L,), jnp.int32)],
                               jnp.broadcast_to(final, (_L,)), mask=lane0)
            pltpu.sync_copy(out_v, out_hbm)

    return sc_loss


def kernel(pred_next_dt, next_dt, dt, offsets):
    tau = lax.stop_gradient(pred_next_dt)
    n = tau.shape[0]
    nseg = offsets.shape[0] - 1
    out = _make_sc_call(n, nseg, float(offsets.shape[0]))(
        tau, dt[:, 0], offsets)
    loss = out.reshape(())
    return (loss, loss)
